# baseline (speedup 1.0000x reference)
import functools

import jax
import jax.numpy as jnp
from jax.experimental import pallas as pl
from jax.experimental.pallas import tpu as pltpu

EPS = 1e-07
INF = 100000.0
CTK = 10
N = 20000
G = 64
C = 80
BN = 2000
CH = 1000
NEG_BIG = -1e30
POS_BIG = 1e30


def _cost_iou_kernel(p_ref, pr_ref, db_ref, gt_ref, lb_ref, cost_ref, iou_ref):
    p = p_ref[...]
    log1m = jnp.log1p(-p)
    sml = jnp.sum(log1m, axis=1, keepdims=True)
    cls_iota = jax.lax.broadcasted_iota(jnp.int32, (C, G), 0)
    oh = (cls_iota == lb_ref[...]).astype(jnp.float32)
    pg = jax.lax.dot(p, oh, preferred_element_type=jnp.float32,
                     precision=jax.lax.Precision.HIGHEST)
    logit = jnp.log(pg) - jnp.log1p(-pg)
    cls_cost = -(logit + sml)

    x = pr_ref[:, 0:1]
    y = pr_ref[:, 1:2]
    sx = pr_ref[:, 2:3]
    sy = pr_ref[:, 3:4]
    gx1 = gt_ref[0:1, :]
    gy1 = gt_ref[1:2, :]
    gx2 = gt_ref[2:3, :]
    gy2 = gt_ref[3:4, :]
    in_gts = (((x - gx1) > 0) & ((y - gy1) > 0)
              & ((gx2 - x) > 0) & ((gy2 - y) > 0))
    gcx = (gx1 + gx2) * 0.5
    gcy = (gy1 + gy2) * 0.5
    r = 2.5
    in_cts = (((x - (gcx - r * sx)) > 0) & ((y - (gcy - r * sy)) > 0)
              & (((gcx + r * sx) - x) > 0) & (((gcy + r * sy) - y) > 0))
    valid = (jnp.any(in_gts, axis=1, keepdims=True)
             | jnp.any(in_cts, axis=1, keepdims=True))
    in_bc = in_gts & in_cts

    dx1 = db_ref[:, 0:1]
    dy1 = db_ref[:, 1:2]
    dx2 = db_ref[:, 2:3]
    dy2 = db_ref[:, 3:4]
    lt_x = jnp.maximum(dx1, gx1)
    lt_y = jnp.maximum(dy1, gy1)
    rb_x = jnp.minimum(dx2, gx2)
    rb_y = jnp.minimum(dy2, gy2)
    w = jnp.clip(rb_x - lt_x, 0.0, None)
    h = jnp.clip(rb_y - lt_y, 0.0, None)
    overlap = w * h
    area_a = (dx2 - dx1) * (dy2 - dy1)
    area_b = (gx2 - gx1) * (gy2 - gy1)
    union = area_a + area_b - overlap
    iou = overlap / jnp.maximum(union, EPS)
    iou = jnp.where(valid, iou, 0.0)

    cost = (cls_cost + 3.0 * (-jnp.log(iou + EPS))
            + jnp.where(in_bc, 0.0, INF)
            + jnp.where(valid, 0.0, 10.0 * INF))
    cost_ref[...] = cost
    iou_ref[...] = iou


def _assign_kernel(cost_ref, iou_ref, gi_ref, lab_ref, giou_ref):
    nch = N // CH

    def iou_pass(i, carry):
        thr, cnt, u = carry

        def chunk(c, acc):
            mx, ct = acc
            x = iou_ref[pl.ds(c * CH, CH), :]
            masked = jnp.where(x < thr, x, NEG_BIG)
            mx = jnp.maximum(mx, jnp.max(masked, axis=0, keepdims=True))
            ct = ct + jnp.sum((x == thr).astype(jnp.int32), axis=0,
                              keepdims=True)
            return mx, ct

        mx0 = jnp.full((1, G), NEG_BIG, jnp.float32)
        ct0 = jnp.zeros((1, G), jnp.int32)
        mx, ct = jax.lax.fori_loop(0, nch, chunk, (mx0, ct0))
        cnt = cnt + ct
        u = jnp.where(cnt < CTK, mx, u)
        return mx, cnt, u

    thr0 = jnp.full((1, G), POS_BIG, jnp.float32)
    cnt0 = jnp.zeros((1, G), jnp.int32)
    u0 = jnp.zeros((1, G), jnp.float32)
    _, _, u = jax.lax.fori_loop(0, CTK, iou_pass, (thr0, cnt0, u0))

    def sum_chunk(c, acc):
        s, cg = acc
        x = iou_ref[pl.ds(c * CH, CH), :]
        gt_u = x > u
        s = s + jnp.sum(jnp.where(gt_u, x, 0.0), axis=0, keepdims=True)
        cg = cg + jnp.sum(gt_u.astype(jnp.int32), axis=0, keepdims=True)
        return s, cg

    s0 = jnp.zeros((1, G), jnp.float32)
    s, cg = jax.lax.fori_loop(0, nch, sum_chunk,
                              (s0, jnp.zeros((1, G), jnp.int32)))
    sum10 = s + (CTK - cg).astype(jnp.float32) * u
    dyn_k = jnp.maximum(sum10.astype(jnp.int32), 1)

    def cost_pass(i, carry):
        thr, cnt, t = carry

        def chunk(c, acc):
            mn, ct = acc
            x = cost_ref[pl.ds(c * CH, CH), :]
            masked = jnp.where(x > thr, x, POS_BIG)
            mn = jnp.minimum(mn, jnp.min(masked, axis=0, keepdims=True))
            ct = ct + jnp.sum((x == thr).astype(jnp.int32), axis=0,
                              keepdims=True)
            return mn, ct

        mn0 = jnp.full((1, G), POS_BIG, jnp.float32)
        ct0 = jnp.zeros((1, G), jnp.int32)
        mn, ct = jax.lax.fori_loop(0, nch, chunk, (mn0, ct0))
        cnt = cnt + ct
        t = jnp.where(cnt < dyn_k, mn, t)
        return mn, cnt, t

    tthr0 = jnp.full((1, G), NEG_BIG, jnp.float32)
    t0 = jnp.full((1, G), POS_BIG, jnp.float32)
    _, _, t = jax.lax.fori_loop(0, CTK, cost_pass,
                                (tthr0, jnp.zeros((1, G), jnp.int32), t0))

    def out_chunk(c, _):
        x = cost_ref[pl.ds(c * CH, CH), :]
        io = iou_ref[pl.ds(c * CH, CH), :]
        sel = x <= t
        nsel = jnp.sum(sel.astype(jnp.int32), axis=1, keepdims=True)
        gio = jax.lax.broadcasted_iota(jnp.int32, (CH, G), 1)
        rowmin = jnp.min(x, axis=1, keepdims=True)
        amin = jnp.min(jnp.where(x == rowmin, gio, 2 ** 30), axis=1,
                       keepdims=True)
        g1 = jnp.min(jnp.where(sel, gio, 2 ** 30), axis=1, keepdims=True)
        assigned = jnp.where(nsel >= 2, amin, g1)
        fg = nsel > 0
        iou_sel = jnp.sum(jnp.where(gio == assigned, io, 0.0), axis=1,
                          keepdims=True)
        gi_ref[pl.ds(c * CH, CH), :] = jnp.where(fg, assigned, 0)
        lab_ref[pl.ds(c * CH, CH), :] = jnp.where(fg, 1, -1)
        giou_ref[pl.ds(c * CH, CH), :] = jnp.where(fg, iou_sel, -INF)
        return 0

    jax.lax.fori_loop(0, nch, out_chunk, 0)


@jax.jit
def kernel(pred_scores, priors, decoded_bboxes, gt_bboxes, gt_labels):
    gt_t = gt_bboxes.T
    lb = gt_labels.reshape(1, G)

    cost, iou = pl.pallas_call(
        _cost_iou_kernel,
        grid=(N // BN,),
        in_specs=[
            pl.BlockSpec((BN, C), lambda i: (i, 0)),
            pl.BlockSpec((BN, 4), lambda i: (i, 0)),
            pl.BlockSpec((BN, 4), lambda i: (i, 0)),
            pl.BlockSpec((4, G), lambda i: (0, 0)),
            pl.BlockSpec((1, G), lambda i: (0, 0)),
        ],
        out_specs=[
            pl.BlockSpec((BN, G), lambda i: (i, 0)),
            pl.BlockSpec((BN, G), lambda i: (i, 0)),
        ],
        out_shape=[
            jax.ShapeDtypeStruct((N, G), jnp.float32),
            jax.ShapeDtypeStruct((N, G), jnp.float32),
        ],
    )(pred_scores, priors, decoded_bboxes, gt_t, lb)

    gi, lab, giou = pl.pallas_call(
        _assign_kernel,
        out_shape=[
            jax.ShapeDtypeStruct((N, 1), jnp.int32),
            jax.ShapeDtypeStruct((N, 1), jnp.int32),
            jax.ShapeDtypeStruct((N, 1), jnp.float32),
        ],
    )(cost, iou)

    return gi.reshape(N), lab.reshape(N), giou.reshape(N)


# speedup vs baseline: 1.3916x; 1.3916x over previous
import functools

import jax
import jax.numpy as jnp
from jax import lax
from jax.experimental import pallas as pl
from jax.experimental.pallas import tpu as pltpu
from jax.experimental.pallas import tpu_sc as plsc

EPS = 1e-07
INF = 100000.0
CTK = 10
N = 20000
NP = 20480
G = 64
C = 80
BN = 2560
PAD_COST = 1e9
NEG_BIG = -1e30
POS_BIG = 1e30
HI = jax.lax.Precision.HIGHEST


def _cost_iou_kernel(p_ref, pr_ref, db_ref, gt_ref, lb_ref, cost_ref, iou_ref):
    p = p_ref[...]
    log1m = jnp.log1p(-p)
    ones = jnp.ones((1, C), jnp.float32)
    smlT = lax.dot_general(ones, log1m, (((1,), (1,)), ((), ())),
                           precision=HI)
    oh = (jax.lax.broadcasted_iota(jnp.int32, (G, C), 1)
          == lb_ref[...]).astype(jnp.float32)
    pgT = lax.dot_general(oh, p, (((1,), (1,)), ((), ())),
                          precision=HI)
    logitT = jnp.log(pgT) - jnp.log1p(-pgT)
    clsT = -(logitT + smlT)

    x = pr_ref[0:1, :]
    y = pr_ref[1:2, :]
    sx = pr_ref[2:3, :]
    sy = pr_ref[3:4, :]
    gx1 = gt_ref[:, 0:1]
    gy1 = gt_ref[:, 1:2]
    gx2 = gt_ref[:, 2:3]
    gy2 = gt_ref[:, 3:4]
    in_gts = (((x - gx1) > 0) & ((y - gy1) > 0)
              & ((gx2 - x) > 0) & ((gy2 - y) > 0))
    gcx = (gx1 + gx2) * 0.5
    gcy = (gy1 + gy2) * 0.5
    r = 2.5
    in_cts = (((x - (gcx - r * sx)) > 0) & ((y - (gcy - r * sy)) > 0)
              & (((gcx + r * sx) - x) > 0) & (((gcy + r * sy) - y) > 0))
    valid = (jnp.any(in_gts, axis=0, keepdims=True)
             | jnp.any(in_cts, axis=0, keepdims=True))
    in_bc = in_gts & in_cts

    dx1 = db_ref[0:1, :]
    dy1 = db_ref[1:2, :]
    dx2 = db_ref[2:3, :]
    dy2 = db_ref[3:4, :]
    w = jnp.clip(jnp.minimum(dx2, gx2) - jnp.maximum(dx1, gx1), 0.0, None)
    h = jnp.clip(jnp.minimum(dy2, gy2) - jnp.maximum(dy1, gy1), 0.0, None)
    overlap = w * h
    area_a = (dx2 - dx1) * (dy2 - dy1)
    area_b = (gx2 - gx1) * (gy2 - gy1)
    union = area_a + area_b - overlap
    iou = overlap / jnp.maximum(union, EPS)
    iou = jnp.where(valid, iou, 0.0)

    cost = (clsT + 3.0 * (-jnp.log(iou + EPS))
            + jnp.where(in_bc, 0.0, INF)
            + jnp.where(valid, 0.0, 10.0 * INF))
    lane = (jax.lax.broadcasted_iota(jnp.int32, (1, BN), 1)
            + pl.program_id(0) * BN)
    cost = jnp.where(lane >= N, PAD_COST, cost)
    cost_ref[...] = cost
    iou_ref[...] = iou


def _pass_a(pred_scores, priors, decoded_bboxes, gt_bboxes, gt_labels):
    return pl.pallas_call(
        _cost_iou_kernel,
        grid=(NP // BN,),
        in_specs=[
            pl.BlockSpec((BN, C), lambda i: (i, 0)),
            pl.BlockSpec((4, BN), lambda i: (0, i)),
            pl.BlockSpec((4, BN), lambda i: (0, i)),
            pl.BlockSpec((G, 4), lambda i: (0, 0)),
            pl.BlockSpec((G, 1), lambda i: (0, 0)),
        ],
        out_specs=[
            pl.BlockSpec((G, BN), lambda i: (0, i)),
            pl.BlockSpec((G, BN), lambda i: (0, i)),
        ],
        out_shape=[
            jax.ShapeDtypeStruct((G, NP), jnp.float32),
            jax.ShapeDtypeStruct((G, NP), jnp.float32),
        ],
    )(jnp.pad(pred_scores, ((0, NP - N), (0, 0)), constant_values=0.5),
      jnp.pad(priors, ((0, NP - N), (0, 0))).T,
      jnp.pad(decoded_bboxes, ((0, NP - N), (0, 0))).T,
      gt_bboxes, gt_labels.reshape(G, 1))


_VCH = 80
_NSTEP = NP // _VCH


def _lane(vec, i):
    io = jax.lax.iota(jnp.int32, 16)
    return jnp.max(jnp.where(io == i, vec, NEG_BIG))


def _merge_hi(best, vs):
    s = [lax.sort(v) for v in vs]
    t01 = lax.sort(jnp.maximum(s[0], lax.rev(s[1], (0,))))
    t23 = lax.sort(jnp.maximum(s[2], lax.rev(s[3], (0,))))
    t = lax.sort(jnp.maximum(t01, lax.rev(t23, (0,))))
    t = lax.sort(jnp.maximum(t, lax.rev(s[4], (0,))))
    return lax.sort(jnp.maximum(best, lax.rev(t, (0,))))


def _merge_lo(best, vs):
    s = [lax.sort(v) for v in vs]
    t01 = lax.sort(jnp.minimum(s[0], lax.rev(s[1], (0,))))
    t23 = lax.sort(jnp.minimum(s[2], lax.rev(s[3], (0,))))
    t = lax.sort(jnp.minimum(t01, lax.rev(t23, (0,))))
    t = lax.sort(jnp.minimum(t, lax.rev(s[4], (0,))))
    return lax.sort(jnp.minimum(best, lax.rev(t, (0,))))


def _scan_top16_hi(buf, keep):
    def step(c, carry):
        best, thr = carry
        base = c * _VCH
        vs = [buf[pl.ds(base + 16 * i, 16)] for i in range(5)]
        m = jnp.max(jnp.maximum(
            jnp.maximum(jnp.maximum(vs[0], vs[1]), jnp.maximum(vs[2], vs[3])),
            vs[4]))

        def do(_):
            nb = _merge_hi(best, vs)
            return nb, _lane(nb, 16 - keep)

        return lax.cond(m > thr, do, lambda _: (best, thr), 0)

    best0 = jnp.full((16,), NEG_BIG, jnp.float32)
    best, _ = lax.fori_loop(0, _NSTEP, step, (best0, NEG_BIG))
    return best


def _scan_top16_lo(buf, keep):
    def step(c, carry):
        best, thr = carry
        base = c * _VCH
        vs = [buf[pl.ds(base + 16 * i, 16)] for i in range(5)]
        m = jnp.min(jnp.minimum(
            jnp.minimum(jnp.minimum(vs[0], vs[1]), jnp.minimum(vs[2], vs[3])),
            vs[4]))

        def do(_):
            nb = _merge_lo(best, vs)
            return nb, _lane(nb, keep - 1)

        return lax.cond(m < thr, do, lambda _: (best, thr), 0)

    best0 = jnp.full((16,), POS_BIG, jnp.float32)
    best, _ = lax.fori_loop(0, _NSTEP, step, (best0, POS_BIG))
    return best


def _sc_thresholds(iou_t, cost_t):
    mesh = plsc.VectorSubcoreMesh(core_axis_name="c", subcore_axis_name="s")

    @functools.partial(
        pl.kernel,
        mesh=mesh,
        compiler_params=pltpu.CompilerParams(needs_layout_passes=False),
        out_type=jax.ShapeDtypeStruct((G, 16), jnp.float32),
        scratch_types=[
            pltpu.VMEM((NP,), jnp.float32),
            pltpu.VMEM((NP,), jnp.float32),
            pltpu.VMEM((16,), jnp.float32),
        ],
    )
    def sck(iou_hbm, cost_hbm, t_hbm, bufa, bufb, tbuf):
        wid = lax.axis_index("s") * 2 + lax.axis_index("c")
        for half in range(2):
            g = wid + half * 32
            pltpu.sync_copy(iou_hbm.at[g], bufa)
            best_i = _scan_top16_hi(bufa, CTK)
            io16 = jax.lax.iota(jnp.int32, 16)
            sum10 = jnp.sum(jnp.where(io16 >= 16 - CTK, best_i, 0.0))
            ki = sum10.astype(jnp.int32)
            ki = ki - (ki.astype(jnp.float32) > sum10).astype(jnp.int32)
            k = jnp.maximum(ki, 1)
            pltpu.sync_copy(cost_hbm.at[g], bufb)
            best_c = _scan_top16_lo(bufb, CTK)
            t = jnp.max(jnp.where(io16 == k - 1, best_c, NEG_BIG))
            tbuf[...] = jnp.full((16,), 1.0, jnp.float32) * t
            pltpu.sync_copy(tbuf, t_hbm.at[g])

    return sck(iou_t, cost_t)


def _assign_kernel(cost_ref, iou_ref, t_ref, gi_ref, lab_ref, giou_ref):
    x = cost_ref[...]
    io = iou_ref[...]
    t = t_ref[...]
    sel = x <= t
    nsel = jnp.sum(sel.astype(jnp.int32), axis=0, keepdims=True)
    gio = jax.lax.broadcasted_iota(jnp.int32, (G, BN), 0)
    colmin = jnp.min(x, axis=0, keepdims=True)
    amin = jnp.min(jnp.where(x == colmin, gio, 2 ** 30), axis=0,
                   keepdims=True)
    g1 = jnp.min(jnp.where(sel, gio, 2 ** 30), axis=0, keepdims=True)
    assigned = jnp.where(nsel >= 2, amin, g1)
    fg = nsel > 0
    iou_sel = jnp.sum(jnp.where(gio == assigned, io, 0.0), axis=0,
                      keepdims=True)
    gi_ref[...] = jnp.where(fg, assigned, 0)
    lab_ref[...] = jnp.where(fg, 1, -1)
    giou_ref[...] = jnp.where(fg, iou_sel, -INF)


def _pass_b(cost_t, iou_t, t_col):
    return pl.pallas_call(
        _assign_kernel,
        grid=(NP // BN,),
        in_specs=[
            pl.BlockSpec((G, BN), lambda i: (0, i)),
            pl.BlockSpec((G, BN), lambda i: (0, i)),
            pl.BlockSpec((G, 1), lambda i: (0, 0)),
        ],
        out_specs=[
            pl.BlockSpec((1, BN), lambda i: (0, i)),
            pl.BlockSpec((1, BN), lambda i: (0, i)),
            pl.BlockSpec((1, BN), lambda i: (0, i)),
        ],
        out_shape=[
            jax.ShapeDtypeStruct((1, NP), jnp.int32),
            jax.ShapeDtypeStruct((1, NP), jnp.int32),
            jax.ShapeDtypeStruct((1, NP), jnp.float32),
        ],
    )(cost_t, iou_t, t_col)


@jax.jit
def kernel(pred_scores, priors, decoded_bboxes, gt_bboxes, gt_labels):
    cost_t, iou_t = _pass_a(pred_scores, priors, decoded_bboxes,
                            gt_bboxes, gt_labels)
    t16 = _sc_thresholds(iou_t, cost_t)
    gi, lab, giou = _pass_b(cost_t, iou_t, t16[:, 0:1])
    return gi.reshape(NP)[:N], lab.reshape(NP)[:N], giou.reshape(NP)[:N]


# speedup vs baseline: 1.3916x; 1.0000x over previous
import functools

import jax
import jax.numpy as jnp
from jax import lax
from jax.experimental import pallas as pl
from jax.experimental.pallas import tpu as pltpu
from jax.experimental.pallas import tpu_sc as plsc

EPS = 1e-07
INF = 100000.0
CTK = 10
N = 20000
NP = 20480
G = 64
C = 80
BN = 2560
PAD_COST = 1e9
NEG_BIG = -1e30
POS_BIG = 1e30
HI = jax.lax.Precision.HIGHEST


def _cost_iou_kernel(p_ref, pr_ref, db_ref, gt_ref, lb_ref, cost_ref, iou_ref):
    p = p_ref[...]
    log1m = jnp.log1p(-p)
    ones = jnp.ones((1, C), jnp.float32)
    smlT = lax.dot_general(ones, log1m, (((1,), (1,)), ((), ())),
                           precision=HI)
    oh = (jax.lax.broadcasted_iota(jnp.int32, (G, C), 1)
          == lb_ref[...]).astype(jnp.float32)
    pgT = lax.dot_general(oh, p, (((1,), (1,)), ((), ())),
                          precision=HI)
    logitT = jnp.log(pgT) - jnp.log1p(-pgT)
    clsT = -(logitT + smlT)

    x = pr_ref[0:1, :]
    y = pr_ref[1:2, :]
    sx = pr_ref[2:3, :]
    sy = pr_ref[3:4, :]
    gx1 = gt_ref[:, 0:1]
    gy1 = gt_ref[:, 1:2]
    gx2 = gt_ref[:, 2:3]
    gy2 = gt_ref[:, 3:4]
    in_gts = (((x - gx1) > 0) & ((y - gy1) > 0)
              & ((gx2 - x) > 0) & ((gy2 - y) > 0))
    gcx = (gx1 + gx2) * 0.5
    gcy = (gy1 + gy2) * 0.5
    r = 2.5
    in_cts = (((x - (gcx - r * sx)) > 0) & ((y - (gcy - r * sy)) > 0)
              & (((gcx + r * sx) - x) > 0) & (((gcy + r * sy) - y) > 0))
    valid = (jnp.any(in_gts, axis=0, keepdims=True)
             | jnp.any(in_cts, axis=0, keepdims=True))
    in_bc = in_gts & in_cts

    dx1 = db_ref[0:1, :]
    dy1 = db_ref[1:2, :]
    dx2 = db_ref[2:3, :]
    dy2 = db_ref[3:4, :]
    w = jnp.clip(jnp.minimum(dx2, gx2) - jnp.maximum(dx1, gx1), 0.0, None)
    h = jnp.clip(jnp.minimum(dy2, gy2) - jnp.maximum(dy1, gy1), 0.0, None)
    overlap = w * h
    area_a = (dx2 - dx1) * (dy2 - dy1)
    area_b = (gx2 - gx1) * (gy2 - gy1)
    union = area_a + area_b - overlap
    iou = overlap / jnp.maximum(union, EPS)
    iou = jnp.where(valid, iou, 0.0)

    cost = (clsT + 3.0 * (-jnp.log(iou + EPS))
            + jnp.where(in_bc, 0.0, INF)
            + jnp.where(valid, 0.0, 10.0 * INF))
    lane = (jax.lax.broadcasted_iota(jnp.int32, (1, BN), 1)
            + pl.program_id(0) * BN)
    cost = jnp.where(lane >= N, PAD_COST, cost)
    cost_ref[...] = cost
    iou_ref[...] = iou


def _pass_a(pred_scores, priors, decoded_bboxes, gt_bboxes, gt_labels):
    return pl.pallas_call(
        _cost_iou_kernel,
        grid=(NP // BN,),
        in_specs=[
            pl.BlockSpec((BN, C), lambda i: (i, 0)),
            pl.BlockSpec((4, BN), lambda i: (0, i)),
            pl.BlockSpec((4, BN), lambda i: (0, i)),
            pl.BlockSpec((G, 4), lambda i: (0, 0)),
            pl.BlockSpec((G, 1), lambda i: (0, 0)),
        ],
        out_specs=[
            pl.BlockSpec((G, BN), lambda i: (0, i)),
            pl.BlockSpec((G, BN), lambda i: (0, i)),
        ],
        out_shape=[
            jax.ShapeDtypeStruct((G, NP), jnp.float32),
            jax.ShapeDtypeStruct((G, NP), jnp.float32),
        ],
    )(jnp.pad(pred_scores, ((0, NP - N), (0, 0)), constant_values=0.5),
      jnp.pad(priors, ((0, NP - N), (0, 0))).T,
      jnp.pad(decoded_bboxes, ((0, NP - N), (0, 0))).T,
      gt_bboxes, gt_labels.reshape(G, 1))


_VCH = 80
_NSTEP = NP // _VCH


def _lane(vec, i):
    io = jax.lax.iota(jnp.int32, 16)
    return jnp.max(jnp.where(io == i, vec, NEG_BIG))


def _merge_hi(best, vs):
    s = [lax.sort(v) for v in vs]
    t01 = lax.sort(jnp.maximum(s[0], lax.rev(s[1], (0,))))
    t23 = lax.sort(jnp.maximum(s[2], lax.rev(s[3], (0,))))
    t = lax.sort(jnp.maximum(t01, lax.rev(t23, (0,))))
    t = lax.sort(jnp.maximum(t, lax.rev(s[4], (0,))))
    return lax.sort(jnp.maximum(best, lax.rev(t, (0,))))


def _merge_lo(best, vs):
    s = [lax.sort(v) for v in vs]
    t01 = lax.sort(jnp.minimum(s[0], lax.rev(s[1], (0,))))
    t23 = lax.sort(jnp.minimum(s[2], lax.rev(s[3], (0,))))
    t = lax.sort(jnp.minimum(t01, lax.rev(t23, (0,))))
    t = lax.sort(jnp.minimum(t, lax.rev(s[4], (0,))))
    return lax.sort(jnp.minimum(best, lax.rev(t, (0,))))


def _scan_top16_hi(buf, keep):
    def step(c, carry):
        best, thr = carry
        base = c * _VCH
        vs = [buf[pl.ds(base + 16 * i, 16)] for i in range(5)]
        m = jnp.max(jnp.maximum(
            jnp.maximum(jnp.maximum(vs[0], vs[1]), jnp.maximum(vs[2], vs[3])),
            vs[4]))

        def do(_):
            nb = _merge_hi(best, vs)
            return nb, _lane(nb, 16 - keep)

        return lax.cond(m > thr, do, lambda _: (best, thr), 0)

    best0 = jnp.full((16,), NEG_BIG, jnp.float32)
    best, _ = lax.fori_loop(0, _NSTEP, step, (best0, NEG_BIG))
    return best


def _scan_top16_lo(buf, keep):
    def step(c, carry):
        best, thr = carry
        base = c * _VCH
        vs = [buf[pl.ds(base + 16 * i, 16)] for i in range(5)]
        m = jnp.min(jnp.minimum(
            jnp.minimum(jnp.minimum(vs[0], vs[1]), jnp.minimum(vs[2], vs[3])),
            vs[4]))

        def do(_):
            nb = _merge_lo(best, vs)
            return nb, _lane(nb, keep - 1)

        return lax.cond(m < thr, do, lambda _: (best, thr), 0)

    best0 = jnp.full((16,), POS_BIG, jnp.float32)
    best, _ = lax.fori_loop(0, _NSTEP, step, (best0, POS_BIG))
    return best


def _sc_thresholds(iou_t, cost_t):
    mesh = plsc.VectorSubcoreMesh(core_axis_name="c", subcore_axis_name="s")

    @functools.partial(
        pl.kernel,
        mesh=mesh,
        compiler_params=pltpu.CompilerParams(needs_layout_passes=False,
                                             use_tc_tiling_on_sc=True),
        out_type=jax.ShapeDtypeStruct((G, 16), jnp.float32),
        scratch_types=[
            pltpu.VMEM((NP,), jnp.float32),
            pltpu.VMEM((NP,), jnp.float32),
            pltpu.VMEM((16,), jnp.float32),
        ],
    )
    def sck(iou_hbm, cost_hbm, t_hbm, bufa, bufb, tbuf):
        wid = lax.axis_index("s") * 2 + lax.axis_index("c")
        for half in range(2):
            g = wid + half * 32
            pltpu.sync_copy(iou_hbm.at[g], bufa)
            best_i = _scan_top16_hi(bufa, CTK)
            io16 = jax.lax.iota(jnp.int32, 16)
            sum10 = jnp.sum(jnp.where(io16 >= 16 - CTK, best_i, 0.0))
            ki = sum10.astype(jnp.int32)
            ki = ki - (ki.astype(jnp.float32) > sum10).astype(jnp.int32)
            k = jnp.maximum(ki, 1)
            pltpu.sync_copy(cost_hbm.at[g], bufb)
            best_c = _scan_top16_lo(bufb, CTK)
            t = jnp.max(jnp.where(io16 == k - 1, best_c, NEG_BIG))
            tbuf[...] = jnp.full((16,), 1.0, jnp.float32) * t
            pltpu.sync_copy(tbuf, t_hbm.at[g])

    return sck(iou_t, cost_t)


def _assign_kernel(cost_ref, iou_ref, t_ref, gi_ref, lab_ref, giou_ref):
    x = cost_ref[...]
    io = iou_ref[...]
    t = t_ref[...]
    sel = x <= t
    nsel = jnp.sum(sel.astype(jnp.int32), axis=0, keepdims=True)
    gio = jax.lax.broadcasted_iota(jnp.int32, (G, BN), 0)
    colmin = jnp.min(x, axis=0, keepdims=True)
    amin = jnp.min(jnp.where(x == colmin, gio, 2 ** 30), axis=0,
                   keepdims=True)
    g1 = jnp.min(jnp.where(sel, gio, 2 ** 30), axis=0, keepdims=True)
    assigned = jnp.where(nsel >= 2, amin, g1)
    fg = nsel > 0
    iou_sel = jnp.sum(jnp.where(gio == assigned, io, 0.0), axis=0,
                      keepdims=True)
    gi_ref[...] = jnp.where(fg, assigned, 0)
    lab_ref[...] = jnp.where(fg, 1, -1)
    giou_ref[...] = jnp.where(fg, iou_sel, -INF)


def _pass_b(cost_t, iou_t, t_col):
    return pl.pallas_call(
        _assign_kernel,
        grid=(NP // BN,),
        in_specs=[
            pl.BlockSpec((G, BN), lambda i: (0, i)),
            pl.BlockSpec((G, BN), lambda i: (0, i)),
            pl.BlockSpec((G, 1), lambda i: (0, 0)),
        ],
        out_specs=[
            pl.BlockSpec((1, BN), lambda i: (0, i)),
            pl.BlockSpec((1, BN), lambda i: (0, i)),
            pl.BlockSpec((1, BN), lambda i: (0, i)),
        ],
        out_shape=[
            jax.ShapeDtypeStruct((1, NP), jnp.int32),
            jax.ShapeDtypeStruct((1, NP), jnp.int32),
            jax.ShapeDtypeStruct((1, NP), jnp.float32),
        ],
    )(cost_t, iou_t, t_col)


@jax.jit
def kernel(pred_scores, priors, decoded_bboxes, gt_bboxes, gt_labels):
    cost_t, iou_t = _pass_a(pred_scores, priors, decoded_bboxes,
                            gt_bboxes, gt_labels)
    t16 = _sc_thresholds(iou_t, cost_t)
    gi, lab, giou = _pass_b(cost_t, iou_t, t16[:, 0:1])
    return gi.reshape(NP)[:N], lab.reshape(NP)[:N], giou.reshape(NP)[:N]


# speedup vs baseline: 1.8775x; 1.3492x over previous
import functools

import jax
import jax.numpy as jnp
from jax import lax
from jax.experimental import pallas as pl
from jax.experimental.pallas import tpu as pltpu
from jax.experimental.pallas import tpu_sc as plsc

EPS = 1e-07
INF = 100000.0
CTK = 10
N = 20000
NP = 20480
G = 64
C = 80
BN = 2560
PAD_COST = 1e9
NEG_BIG = -1e30
POS_BIG = 1e30
HI = jax.lax.Precision.HIGHEST


def _cost_iou_kernel(p_ref, pr_ref, db_ref, gt_ref, lb_ref, cost_ref, iou_ref):
    p = p_ref[...]
    log1m = jnp.log1p(-p)
    ones = jnp.ones((1, C), jnp.float32)
    smlT = lax.dot_general(ones, log1m, (((1,), (1,)), ((), ())),
                           precision=HI)
    oh = (jax.lax.broadcasted_iota(jnp.int32, (G, C), 1)
          == lb_ref[...]).astype(jnp.float32)
    pgT = lax.dot_general(oh, p, (((1,), (1,)), ((), ())),
                          precision=HI)
    logitT = jnp.log(pgT) - jnp.log1p(-pgT)
    clsT = -(logitT + smlT)

    x = pr_ref[0:1, :]
    y = pr_ref[1:2, :]
    sx = pr_ref[2:3, :]
    sy = pr_ref[3:4, :]
    gx1 = gt_ref[:, 0:1]
    gy1 = gt_ref[:, 1:2]
    gx2 = gt_ref[:, 2:3]
    gy2 = gt_ref[:, 3:4]
    in_gts = (((x - gx1) > 0) & ((y - gy1) > 0)
              & ((gx2 - x) > 0) & ((gy2 - y) > 0))
    gcx = (gx1 + gx2) * 0.5
    gcy = (gy1 + gy2) * 0.5
    r = 2.5
    in_cts = (((x - (gcx - r * sx)) > 0) & ((y - (gcy - r * sy)) > 0)
              & (((gcx + r * sx) - x) > 0) & (((gcy + r * sy) - y) > 0))
    valid = (jnp.any(in_gts, axis=0, keepdims=True)
             | jnp.any(in_cts, axis=0, keepdims=True))
    in_bc = in_gts & in_cts

    dx1 = db_ref[0:1, :]
    dy1 = db_ref[1:2, :]
    dx2 = db_ref[2:3, :]
    dy2 = db_ref[3:4, :]
    w = jnp.clip(jnp.minimum(dx2, gx2) - jnp.maximum(dx1, gx1), 0.0, None)
    h = jnp.clip(jnp.minimum(dy2, gy2) - jnp.maximum(dy1, gy1), 0.0, None)
    overlap = w * h
    area_a = (dx2 - dx1) * (dy2 - dy1)
    area_b = (gx2 - gx1) * (gy2 - gy1)
    union = area_a + area_b - overlap
    iou = overlap / jnp.maximum(union, EPS)
    iou = jnp.where(valid, iou, 0.0)

    cost = (clsT + 3.0 * (-jnp.log(iou + EPS))
            + jnp.where(in_bc, 0.0, INF)
            + jnp.where(valid, 0.0, 10.0 * INF))
    lane = (jax.lax.broadcasted_iota(jnp.int32, (1, BN), 1)
            + pl.program_id(0) * BN)
    cost = jnp.where(lane >= N, PAD_COST, cost)
    cost_ref[...] = cost
    iou_ref[...] = iou


def _pass_a(pred_scores, priors, decoded_bboxes, gt_bboxes, gt_labels):
    return pl.pallas_call(
        _cost_iou_kernel,
        grid=(NP // BN,),
        in_specs=[
            pl.BlockSpec((BN, C), lambda i: (i, 0)),
            pl.BlockSpec((4, BN), lambda i: (0, i)),
            pl.BlockSpec((4, BN), lambda i: (0, i)),
            pl.BlockSpec((G, 4), lambda i: (0, 0)),
            pl.BlockSpec((G, 1), lambda i: (0, 0)),
        ],
        out_specs=[
            pl.BlockSpec((G, BN), lambda i: (0, i)),
            pl.BlockSpec((G, BN), lambda i: (0, i)),
        ],
        out_shape=[
            jax.ShapeDtypeStruct((G, NP), jnp.float32),
            jax.ShapeDtypeStruct((G, NP), jnp.float32),
        ],
    )(jnp.pad(pred_scores, ((0, NP - N), (0, 0)), constant_values=0.5),
      jnp.pad(priors, ((0, NP - N), (0, 0))).T,
      jnp.pad(decoded_bboxes, ((0, NP - N), (0, 0))).T,
      gt_bboxes, gt_labels.reshape(G, 1))


_NROW = NP // 128


def _lane(vec, i):
    io = jax.lax.iota(jnp.int32, 16)
    return jnp.max(jnp.where(io == i, vec, NEG_BIG))


def _vtree(vs, op):
    while len(vs) > 1:
        vs = [op(vs[2 * i], vs[2 * i + 1]) for i in range(len(vs) // 2)] \
             + vs[len(vs) // 2 * 2:]
    return vs[0]


def _merge_hi(best, vs):
    s = [lax.sort(v) for v in vs]
    while len(s) > 1:
        s = [lax.sort(jnp.maximum(s[2 * i], lax.rev(s[2 * i + 1], (0,))))
             for i in range(len(s) // 2)] + s[len(s) // 2 * 2:]
    return lax.sort(jnp.maximum(best, lax.rev(s[0], (0,))))


def _merge_lo(best, vs):
    s = [lax.sort(v) for v in vs]
    while len(s) > 1:
        s = [lax.sort(jnp.minimum(s[2 * i], lax.rev(s[2 * i + 1], (0,))))
             for i in range(len(s) // 2)] + s[len(s) // 2 * 2:]
    return lax.sort(jnp.minimum(best, lax.rev(s[0], (0,))))


def _dual_scan(bufa, bufb, keep, hi):
    if hi:
        init, red, cmp, merge = NEG_BIG, jnp.max, (lambda m, t: m > t), _merge_hi
        kl = 16 - keep
    else:
        init, red, cmp, merge = POS_BIG, jnp.min, (lambda m, t: m < t), _merge_lo
        kl = keep - 1
    opv = jnp.maximum if hi else jnp.minimum

    def step(r, carry):
        ba, ta, bb, tb = carry
        va = [bufa[pl.ds(r * 128 + 16 * i, 16)] for i in range(8)]
        vb = [bufb[pl.ds(r * 128 + 16 * i, 16)] for i in range(8)]
        ma = red(_vtree(list(va), opv))
        mb = red(_vtree(list(vb), opv))

        def doa(_):
            nb = merge(ba, va)
            return nb, _lane(nb, kl)

        def dob(_):
            nb = merge(bb, vb)
            return nb, _lane(nb, kl)

        ba, ta = lax.cond(cmp(ma, ta), doa, lambda _: (ba, ta), 0)
        bb, tb = lax.cond(cmp(mb, tb), dob, lambda _: (bb, tb), 0)
        return ba, ta, bb, tb

    b0 = jnp.full((16,), init, jnp.float32)
    ba, _, bb, _ = lax.fori_loop(0, _NROW, step, (b0, init, b0, init))
    return ba, bb


def _trunc_pos(x):
    xi = x.astype(jnp.int32)
    return xi - (xi.astype(jnp.float32) > x).astype(jnp.int32)


def _sc_thresholds(iou_t, cost_t):
    mesh = plsc.VectorSubcoreMesh(core_axis_name="c", subcore_axis_name="s")

    @functools.partial(
        pl.kernel,
        mesh=mesh,
        compiler_params=pltpu.CompilerParams(needs_layout_passes=False),
        out_type=jax.ShapeDtypeStruct((G, 16), jnp.float32),
        scratch_types=[
            pltpu.VMEM((NP,), jnp.float32),
            pltpu.VMEM((NP,), jnp.float32),
            pltpu.VMEM((NP,), jnp.float32),
            pltpu.VMEM((NP,), jnp.float32),
            pltpu.VMEM((16,), jnp.float32),
            pltpu.SemaphoreType.DMA,
            pltpu.SemaphoreType.DMA,
            pltpu.SemaphoreType.DMA,
            pltpu.SemaphoreType.DMA,
        ],
    )
    def sck(iou_hbm, cost_hbm, t_hbm, bi0, bi1, bc0, bc1, tbuf,
            s0, s1, s2, s3):
        wid = lax.axis_index("s") * 2 + lax.axis_index("c")
        g0 = wid
        g1 = wid + 32
        d0 = pltpu.async_copy(iou_hbm.at[g0], bi0, s0)
        d1 = pltpu.async_copy(iou_hbm.at[g1], bi1, s1)
        d2 = pltpu.async_copy(cost_hbm.at[g0], bc0, s2)
        d3 = pltpu.async_copy(cost_hbm.at[g1], bc1, s3)
        d0.wait()
        d1.wait()
        bia, bib = _dual_scan(bi0, bi1, CTK, True)
        io16 = jax.lax.iota(jnp.int32, 16)
        tk = io16 >= 16 - CTK
        k0 = jnp.maximum(_trunc_pos(jnp.sum(jnp.where(tk, bia, 0.0))), 1)
        k1 = jnp.maximum(_trunc_pos(jnp.sum(jnp.where(tk, bib, 0.0))), 1)
        d2.wait()
        d3.wait()
        bca, bcb = _dual_scan(bc0, bc1, CTK, False)
        t0 = jnp.max(jnp.where(io16 == k0 - 1, bca, NEG_BIG))
        t1 = jnp.max(jnp.where(io16 == k1 - 1, bcb, NEG_BIG))
        tbuf[...] = jnp.full((16,), 1.0, jnp.float32) * t0
        pltpu.sync_copy(tbuf, t_hbm.at[g0])
        tbuf[...] = jnp.full((16,), 1.0, jnp.float32) * t1
        pltpu.sync_copy(tbuf, t_hbm.at[g1])

    return sck(iou_t, cost_t)


def _assign_kernel(cost_ref, iou_ref, t_ref, gi_ref, lab_ref, giou_ref):
    x = cost_ref[...]
    io = iou_ref[...]
    t = t_ref[...]
    sel = x <= t
    nsel = jnp.sum(sel.astype(jnp.int32), axis=0, keepdims=True)
    gio = jax.lax.broadcasted_iota(jnp.int32, (G, BN), 0)
    colmin = jnp.min(x, axis=0, keepdims=True)
    amin = jnp.min(jnp.where(x == colmin, gio, 2 ** 30), axis=0,
                   keepdims=True)
    g1 = jnp.min(jnp.where(sel, gio, 2 ** 30), axis=0, keepdims=True)
    assigned = jnp.where(nsel >= 2, amin, g1)
    fg = nsel > 0
    iou_sel = jnp.sum(jnp.where(gio == assigned, io, 0.0), axis=0,
                      keepdims=True)
    gi_ref[...] = jnp.where(fg, assigned, 0)
    lab_ref[...] = jnp.where(fg, 1, -1)
    giou_ref[...] = jnp.where(fg, iou_sel, -INF)


def _pass_b(cost_t, iou_t, t_col):
    return pl.pallas_call(
        _assign_kernel,
        grid=(NP // BN,),
        in_specs=[
            pl.BlockSpec((G, BN), lambda i: (0, i)),
            pl.BlockSpec((G, BN), lambda i: (0, i)),
            pl.BlockSpec((G, 1), lambda i: (0, 0)),
        ],
        out_specs=[
            pl.BlockSpec((1, BN), lambda i: (0, i)),
            pl.BlockSpec((1, BN), lambda i: (0, i)),
            pl.BlockSpec((1, BN), lambda i: (0, i)),
        ],
        out_shape=[
            jax.ShapeDtypeStruct((1, NP), jnp.int32),
            jax.ShapeDtypeStruct((1, NP), jnp.int32),
            jax.ShapeDtypeStruct((1, NP), jnp.float32),
        ],
    )(cost_t, iou_t, t_col)


@jax.jit
def kernel(pred_scores, priors, decoded_bboxes, gt_bboxes, gt_labels):
    cost_t, iou_t = _pass_a(pred_scores, priors, decoded_bboxes,
                            gt_bboxes, gt_labels)
    t16 = _sc_thresholds(iou_t, cost_t)
    gi, lab, giou = _pass_b(cost_t, iou_t, t16[:, 0:1])
    return gi.reshape(NP)[:N], lab.reshape(NP)[:N], giou.reshape(NP)[:N]


# speedup vs baseline: 1.9262x; 1.0259x over previous
import functools

import jax
import jax.numpy as jnp
from jax import lax
from jax.experimental import pallas as pl
from jax.experimental.pallas import tpu as pltpu
from jax.experimental.pallas import tpu_sc as plsc

EPS = 1e-07
INF = 100000.0
CTK = 10
N = 20000
NP = 20480
G = 64
C = 80
BN = 2048
BNR = BN // 128
PAD_COST = 1e9
NEG_BIG = -1e30
POS_BIG = 1e30
HI = jax.lax.Precision.HIGHEST


def _cost_iou_kernel(p_ref, pr_ref, db_ref, gt_ref, lb_ref, cost_ref, iou_ref):
    p = p_ref[...]
    log1m = jnp.log1p(-p)
    ones = jnp.ones((1, C), jnp.float32)
    smlT = lax.dot_general(ones, log1m, (((1,), (1,)), ((), ())),
                           precision=HI)
    oh = (jax.lax.broadcasted_iota(jnp.int32, (G, C), 1)
          == lb_ref[...]).astype(jnp.float32)
    pgT = lax.dot_general(oh, p, (((1,), (1,)), ((), ())),
                          precision=HI)
    logitT = jnp.log(pgT) - jnp.log1p(-pgT)
    clsT = -(logitT + smlT)

    x = pr_ref[0:1, :]
    y = pr_ref[1:2, :]
    sx = pr_ref[2:3, :]
    sy = pr_ref[3:4, :]
    gx1 = gt_ref[:, 0:1]
    gy1 = gt_ref[:, 1:2]
    gx2 = gt_ref[:, 2:3]
    gy2 = gt_ref[:, 3:4]
    in_gts = (((x - gx1) > 0) & ((y - gy1) > 0)
              & ((gx2 - x) > 0) & ((gy2 - y) > 0))
    gcx = (gx1 + gx2) * 0.5
    gcy = (gy1 + gy2) * 0.5
    r = 2.5
    in_cts = (((x - (gcx - r * sx)) > 0) & ((y - (gcy - r * sy)) > 0)
              & (((gcx + r * sx) - x) > 0) & (((gcy + r * sy) - y) > 0))
    valid = (jnp.any(in_gts, axis=0, keepdims=True)
             | jnp.any(in_cts, axis=0, keepdims=True))
    in_bc = in_gts & in_cts

    dx1 = db_ref[0:1, :]
    dy1 = db_ref[1:2, :]
    dx2 = db_ref[2:3, :]
    dy2 = db_ref[3:4, :]
    w = jnp.clip(jnp.minimum(dx2, gx2) - jnp.maximum(dx1, gx1), 0.0, None)
    h = jnp.clip(jnp.minimum(dy2, gy2) - jnp.maximum(dy1, gy1), 0.0, None)
    overlap = w * h
    area_a = (dx2 - dx1) * (dy2 - dy1)
    area_b = (gx2 - gx1) * (gy2 - gy1)
    union = area_a + area_b - overlap
    iou = overlap / jnp.maximum(union, EPS)
    iou = jnp.where(valid, iou, 0.0)

    cost = (clsT + 3.0 * (-jnp.log(iou + EPS))
            + jnp.where(in_bc, 0.0, INF)
            + jnp.where(valid, 0.0, 10.0 * INF))
    lane = (jax.lax.broadcasted_iota(jnp.int32, (1, BN), 1)
            + pl.program_id(0) * BN)
    cost = jnp.where(lane >= N, PAD_COST, cost)
    cost_ref[...] = cost.reshape(G, BNR, 128)
    iou_ref[...] = iou.reshape(G, BNR, 128)


def _pass_a(pred_scores, priors, decoded_bboxes, gt_bboxes, gt_labels):
    return pl.pallas_call(
        _cost_iou_kernel,
        grid=(NP // BN,),
        in_specs=[
            pl.BlockSpec((BN, C), lambda i: (i, 0)),
            pl.BlockSpec((4, BN), lambda i: (0, i)),
            pl.BlockSpec((4, BN), lambda i: (0, i)),
            pl.BlockSpec((G, 4), lambda i: (0, 0)),
            pl.BlockSpec((G, 1), lambda i: (0, 0)),
        ],
        out_specs=[
            pl.BlockSpec((G, BNR, 128), lambda i: (0, i, 0)),
            pl.BlockSpec((G, BNR, 128), lambda i: (0, i, 0)),
        ],
        out_shape=[
            jax.ShapeDtypeStruct((G, NP // 128, 128), jnp.float32),
            jax.ShapeDtypeStruct((G, NP // 128, 128), jnp.float32),
        ],
    )(jnp.pad(pred_scores, ((0, NP - N), (0, 0)), constant_values=0.5),
      jnp.pad(priors, ((0, NP - N), (0, 0))).T,
      jnp.pad(decoded_bboxes, ((0, NP - N), (0, 0))).T,
      gt_bboxes, gt_labels.reshape(G, 1))


_NROW = NP // 128


def _lane(vec, i):
    io = jax.lax.iota(jnp.int32, 16)
    return jnp.max(jnp.where(io == i, vec, NEG_BIG))


def _vtree(vs, op):
    while len(vs) > 1:
        vs = [op(vs[2 * i], vs[2 * i + 1]) for i in range(len(vs) // 2)] \
             + vs[len(vs) // 2 * 2:]
    return vs[0]


def _merge_hi(best, vs):
    s = [lax.sort(v) for v in vs]
    while len(s) > 1:
        s = [lax.sort(jnp.maximum(s[2 * i], lax.rev(s[2 * i + 1], (0,))))
             for i in range(len(s) // 2)] + s[len(s) // 2 * 2:]
    return lax.sort(jnp.maximum(best, lax.rev(s[0], (0,))))


def _merge_lo(best, vs):
    s = [lax.sort(v) for v in vs]
    while len(s) > 1:
        s = [lax.sort(jnp.minimum(s[2 * i], lax.rev(s[2 * i + 1], (0,))))
             for i in range(len(s) // 2)] + s[len(s) // 2 * 2:]
    return lax.sort(jnp.minimum(best, lax.rev(s[0], (0,))))


def _dual_scan(bufa, bufb, keep, hi):
    if hi:
        init, red, cmp, merge = NEG_BIG, jnp.max, (lambda m, t: m > t), _merge_hi
        kl = 16 - keep
    else:
        init, red, cmp, merge = POS_BIG, jnp.min, (lambda m, t: m < t), _merge_lo
        kl = keep - 1
    opv = jnp.maximum if hi else jnp.minimum

    def step(r, carry):
        ba, ta, bb, tb = carry
        va = [bufa[r, pl.ds(16 * i, 16)] for i in range(8)]
        vb = [bufb[r, pl.ds(16 * i, 16)] for i in range(8)]
        ma = red(_vtree(list(va), opv))
        mb = red(_vtree(list(vb), opv))

        def doa(_):
            nb = merge(ba, va)
            return nb, _lane(nb, kl)

        def dob(_):
            nb = merge(bb, vb)
            return nb, _lane(nb, kl)

        ba, ta = lax.cond(cmp(ma, ta), doa, lambda _: (ba, ta), 0)
        bb, tb = lax.cond(cmp(mb, tb), dob, lambda _: (bb, tb), 0)
        return ba, ta, bb, tb

    b0 = jnp.full((16,), init, jnp.float32)
    ba, _, bb, _ = lax.fori_loop(0, _NROW, step, (b0, init, b0, init))
    return ba, bb


def _trunc_pos(x):
    xi = x.astype(jnp.int32)
    return xi - (xi.astype(jnp.float32) > x).astype(jnp.int32)


def _sc_thresholds(iou_t, cost_t):
    mesh = plsc.VectorSubcoreMesh(core_axis_name="c", subcore_axis_name="s")

    @functools.partial(
        pl.kernel,
        mesh=mesh,
        compiler_params=pltpu.CompilerParams(needs_layout_passes=False),
        out_type=jax.ShapeDtypeStruct((G, 16), jnp.float32),
        scratch_types=[
            pltpu.VMEM((_NROW, 128), jnp.float32),
            pltpu.VMEM((_NROW, 128), jnp.float32),
            pltpu.VMEM((_NROW, 128), jnp.float32),
            pltpu.VMEM((_NROW, 128), jnp.float32),
            pltpu.VMEM((16,), jnp.float32),
            pltpu.SemaphoreType.DMA,
            pltpu.SemaphoreType.DMA,
            pltpu.SemaphoreType.DMA,
            pltpu.SemaphoreType.DMA,
        ],
    )
    def sck(iou_hbm, cost_hbm, t_hbm, bi0, bi1, bc0, bc1, tbuf,
            s0, s1, s2, s3):
        wid = lax.axis_index("s") * 2 + lax.axis_index("c")
        g0 = wid
        g1 = wid + 32
        d0 = pltpu.async_copy(iou_hbm.at[g0], bi0, s0)
        d1 = pltpu.async_copy(iou_hbm.at[g1], bi1, s1)
        d2 = pltpu.async_copy(cost_hbm.at[g0], bc0, s2)
        d3 = pltpu.async_copy(cost_hbm.at[g1], bc1, s3)
        d0.wait()
        d1.wait()
        bia, bib = _dual_scan(bi0, bi1, CTK, True)
        io16 = jax.lax.iota(jnp.int32, 16)
        tk = io16 >= 16 - CTK
        k0 = jnp.maximum(_trunc_pos(jnp.sum(jnp.where(tk, bia, 0.0))), 1)
        k1 = jnp.maximum(_trunc_pos(jnp.sum(jnp.where(tk, bib, 0.0))), 1)
        d2.wait()
        d3.wait()
        bca, bcb = _dual_scan(bc0, bc1, CTK, False)
        t0 = jnp.max(jnp.where(io16 == k0 - 1, bca, NEG_BIG))
        t1 = jnp.max(jnp.where(io16 == k1 - 1, bcb, NEG_BIG))
        tbuf[...] = jnp.full((16,), 1.0, jnp.float32) * t0
        pltpu.sync_copy(tbuf, t_hbm.at[g0])
        tbuf[...] = jnp.full((16,), 1.0, jnp.float32) * t1
        pltpu.sync_copy(tbuf, t_hbm.at[g1])

    return sck(iou_t, cost_t)


def _assign_kernel(cost_ref, iou_ref, t_ref, gi_ref, lab_ref, giou_ref):
    x = cost_ref[...]
    io = iou_ref[...]
    t = t_ref[...].reshape(G, 1, 1)
    sel = x <= t
    nsel = jnp.sum(sel.astype(jnp.int32), axis=0, keepdims=True)
    gio = jax.lax.broadcasted_iota(jnp.int32, (G, BNR, 128), 0)
    colmin = jnp.min(x, axis=0, keepdims=True)
    amin = jnp.min(jnp.where(x == colmin, gio, 2 ** 30), axis=0,
                   keepdims=True)
    g1 = jnp.min(jnp.where(sel, gio, 2 ** 30), axis=0, keepdims=True)
    assigned = jnp.where(nsel >= 2, amin, g1)
    fg = nsel > 0
    iou_sel = jnp.sum(jnp.where(gio == assigned, io, 0.0), axis=0,
                      keepdims=True)
    gi_ref[...] = jnp.where(fg, assigned, 0)
    lab_ref[...] = jnp.where(fg, 1, -1)
    giou_ref[...] = jnp.where(fg, iou_sel, -INF)


def _pass_b(cost_t, iou_t, t_col):
    return pl.pallas_call(
        _assign_kernel,
        grid=(NP // BN,),
        in_specs=[
            pl.BlockSpec((G, BNR, 128), lambda i: (0, i, 0)),
            pl.BlockSpec((G, BNR, 128), lambda i: (0, i, 0)),
            pl.BlockSpec((G, 1), lambda i: (0, 0)),
        ],
        out_specs=[
            pl.BlockSpec((1, BNR, 128), lambda i: (0, i, 0)),
            pl.BlockSpec((1, BNR, 128), lambda i: (0, i, 0)),
            pl.BlockSpec((1, BNR, 128), lambda i: (0, i, 0)),
        ],
        out_shape=[
            jax.ShapeDtypeStruct((1, NP // 128, 128), jnp.int32),
            jax.ShapeDtypeStruct((1, NP // 128, 128), jnp.int32),
            jax.ShapeDtypeStruct((1, NP // 128, 128), jnp.float32),
        ],
    )(cost_t, iou_t, t_col)


@jax.jit
def kernel(pred_scores, priors, decoded_bboxes, gt_bboxes, gt_labels):
    cost_t, iou_t = _pass_a(pred_scores, priors, decoded_bboxes,
                            gt_bboxes, gt_labels)
    t16 = _sc_thresholds(iou_t, cost_t)
    gi, lab, giou = _pass_b(cost_t, iou_t, t16[:, 0:1])
    return gi.reshape(NP)[:N], lab.reshape(NP)[:N], giou.reshape(NP)[:N]


# speedup vs baseline: 2.1542x; 1.1184x over previous
import functools

import jax
import jax.numpy as jnp
from jax import lax
from jax.experimental import pallas as pl
from jax.experimental.pallas import tpu as pltpu
from jax.experimental.pallas import tpu_sc as plsc

EPS = 1e-07
INF = 100000.0
CTK = 10
N = 20000
NP = 20480
G = 64
C = 80
BN = 2048
BNR = BN // 128
PAD_COST = 1e9
NEG_BIG = -1e30
POS_BIG = 1e30
HI = jax.lax.Precision.HIGHEST


def _geom(pr_ref, db_ref, gt_ref):
    x = pr_ref[0:1, :]
    y = pr_ref[1:2, :]
    sx = pr_ref[2:3, :]
    sy = pr_ref[3:4, :]
    gx1 = gt_ref[:, 0:1]
    gy1 = gt_ref[:, 1:2]
    gx2 = gt_ref[:, 2:3]
    gy2 = gt_ref[:, 3:4]
    in_gts = (((x - gx1) > 0) & ((y - gy1) > 0)
              & ((gx2 - x) > 0) & ((gy2 - y) > 0))
    gcx = (gx1 + gx2) * 0.5
    gcy = (gy1 + gy2) * 0.5
    r = 2.5
    in_cts = (((x - (gcx - r * sx)) > 0) & ((y - (gcy - r * sy)) > 0)
              & (((gcx + r * sx) - x) > 0) & (((gcy + r * sy) - y) > 0))
    valid = (jnp.any(in_gts, axis=0, keepdims=True)
             | jnp.any(in_cts, axis=0, keepdims=True))
    dx1 = db_ref[0:1, :]
    dy1 = db_ref[1:2, :]
    dx2 = db_ref[2:3, :]
    dy2 = db_ref[3:4, :]
    w = jnp.clip(jnp.minimum(dx2, gx2) - jnp.maximum(dx1, gx1), 0.0, None)
    h = jnp.clip(jnp.minimum(dy2, gy2) - jnp.maximum(dy1, gy1), 0.0, None)
    overlap = w * h
    area_a = (dx2 - dx1) * (dy2 - dy1)
    area_b = (gx2 - gx1) * (gy2 - gy1)
    union = area_a + area_b - overlap
    iou = overlap / jnp.maximum(union, EPS)
    iou = jnp.where(valid, iou, 0.0)
    return iou, in_gts & in_cts, valid


def _iou_kernel(pr_ref, db_ref, gt_ref, iou_ref):
    iou, _, _ = _geom(pr_ref, db_ref, gt_ref)
    iou_ref[...] = iou.reshape(G, BNR, 128)


def _cost_kernel(p_ref, pr_ref, db_ref, gt_ref, lb_ref, cost_ref):
    p = p_ref[...]
    log1m = jnp.log1p(-p)
    ones = jnp.ones((1, C), jnp.float32)
    smlT = lax.dot_general(ones, log1m, (((1,), (1,)), ((), ())),
                           precision=HI)
    oh = (jax.lax.broadcasted_iota(jnp.int32, (G, C), 1)
          == lb_ref[...]).astype(jnp.float32)
    pgT = lax.dot_general(oh, p, (((1,), (1,)), ((), ())),
                          precision=HI)
    logitT = jnp.log(pgT) - jnp.log1p(-pgT)
    clsT = -(logitT + smlT)
    iou, in_bc, valid = _geom(pr_ref, db_ref, gt_ref)
    cost = (clsT + 3.0 * (-jnp.log(iou + EPS))
            + jnp.where(in_bc, 0.0, INF)
            + jnp.where(valid, 0.0, 10.0 * INF))
    lane = (jax.lax.broadcasted_iota(jnp.int32, (1, BN), 1)
            + pl.program_id(0) * BN)
    cost = jnp.where(lane >= N, PAD_COST, cost)
    cost_ref[...] = cost.reshape(G, BNR, 128)


def _prep(pred_scores, priors, decoded_bboxes):
    return (jnp.pad(pred_scores, ((0, NP - N), (0, 0)), constant_values=0.5),
            jnp.pad(priors, ((0, NP - N), (0, 0))).T,
            jnp.pad(decoded_bboxes, ((0, NP - N), (0, 0))).T)


def _pass_a1(pr_t, db_t, gt_bboxes):
    return pl.pallas_call(
        _iou_kernel,
        grid=(NP // BN,),
        in_specs=[
            pl.BlockSpec((4, BN), lambda i: (0, i)),
            pl.BlockSpec((4, BN), lambda i: (0, i)),
            pl.BlockSpec((G, 4), lambda i: (0, 0)),
        ],
        out_specs=pl.BlockSpec((G, BNR, 128), lambda i: (0, i, 0)),
        out_shape=jax.ShapeDtypeStruct((G, NP // 128, 128), jnp.float32),
    )(pr_t, db_t, gt_bboxes)


def _pass_a2(p_pad, pr_t, db_t, gt_bboxes, gt_labels):
    return pl.pallas_call(
        _cost_kernel,
        grid=(NP // BN,),
        in_specs=[
            pl.BlockSpec((BN, C), lambda i: (i, 0)),
            pl.BlockSpec((4, BN), lambda i: (0, i)),
            pl.BlockSpec((4, BN), lambda i: (0, i)),
            pl.BlockSpec((G, 4), lambda i: (0, 0)),
            pl.BlockSpec((G, 1), lambda i: (0, 0)),
        ],
        out_specs=pl.BlockSpec((G, BNR, 128), lambda i: (0, i, 0)),
        out_shape=jax.ShapeDtypeStruct((G, NP // 128, 128), jnp.float32),
    )(p_pad, pr_t, db_t, gt_bboxes, gt_labels.reshape(G, 1))


_NROW = NP // 128


def _lane(vec, i):
    io = jax.lax.iota(jnp.int32, 16)
    return jnp.max(jnp.where(io == i, vec, NEG_BIG))


def _vtree(vs, op):
    while len(vs) > 1:
        vs = [op(vs[2 * i], vs[2 * i + 1]) for i in range(len(vs) // 2)] \
             + vs[len(vs) // 2 * 2:]
    return vs[0]


def _merge_hi(best, vs):
    s = [lax.sort(v) for v in vs]
    while len(s) > 1:
        s = [lax.sort(jnp.maximum(s[2 * i], lax.rev(s[2 * i + 1], (0,))))
             for i in range(len(s) // 2)] + s[len(s) // 2 * 2:]
    return lax.sort(jnp.maximum(best, lax.rev(s[0], (0,))))


def _merge_lo(best, vs):
    s = [lax.sort(v) for v in vs]
    while len(s) > 1:
        s = [lax.sort(jnp.minimum(s[2 * i], lax.rev(s[2 * i + 1], (0,))))
             for i in range(len(s) // 2)] + s[len(s) // 2 * 2:]
    return lax.sort(jnp.minimum(best, lax.rev(s[0], (0,))))


def _dual_scan(bufa, bufb, keep, hi):
    if hi:
        init, red, cmp, merge = NEG_BIG, jnp.max, (lambda m, t: m > t), _merge_hi
        kl = 16 - keep
    else:
        init, red, cmp, merge = POS_BIG, jnp.min, (lambda m, t: m < t), _merge_lo
        kl = keep - 1
    opv = jnp.maximum if hi else jnp.minimum

    def step(r, carry):
        ba, ta, bb, tb = carry
        va = [bufa[r, pl.ds(16 * i, 16)] for i in range(8)]
        vb = [bufb[r, pl.ds(16 * i, 16)] for i in range(8)]
        ma = red(_vtree(list(va), opv))
        mb = red(_vtree(list(vb), opv))

        def doa(_):
            nb = merge(ba, va)
            return nb, _lane(nb, kl)

        def dob(_):
            nb = merge(bb, vb)
            return nb, _lane(nb, kl)

        ba, ta = lax.cond(cmp(ma, ta), doa, lambda _: (ba, ta), 0)
        bb, tb = lax.cond(cmp(mb, tb), dob, lambda _: (bb, tb), 0)
        return ba, ta, bb, tb

    b0 = jnp.full((16,), init, jnp.float32)
    ba, _, bb, _ = lax.fori_loop(0, _NROW, step, (b0, init, b0, init))
    return ba, bb


def _trunc_pos(x):
    xi = x.astype(jnp.int32)
    return xi - (xi.astype(jnp.float32) > x).astype(jnp.int32)


def _sc_dynk(iou_t):
    mesh = plsc.VectorSubcoreMesh(core_axis_name="c", subcore_axis_name="s")

    @functools.partial(
        pl.kernel,
        mesh=mesh,
        compiler_params=pltpu.CompilerParams(needs_layout_passes=False),
        out_type=jax.ShapeDtypeStruct((G, 16), jnp.float32),
        scratch_types=[
            pltpu.VMEM((_NROW, 128), jnp.float32),
            pltpu.VMEM((_NROW, 128), jnp.float32),
            pltpu.VMEM((16,), jnp.float32),
            pltpu.SemaphoreType.DMA,
            pltpu.SemaphoreType.DMA,
        ],
    )
    def sck(iou_hbm, k_hbm, bi0, bi1, tbuf, s0, s1):
        wid = lax.axis_index("s") * 2 + lax.axis_index("c")
        g0 = wid
        g1 = wid + 32
        d0 = pltpu.async_copy(iou_hbm.at[g0], bi0, s0)
        d1 = pltpu.async_copy(iou_hbm.at[g1], bi1, s1)
        d0.wait()
        d1.wait()
        bia, bib = _dual_scan(bi0, bi1, CTK, True)
        io16 = jax.lax.iota(jnp.int32, 16)
        tk = io16 >= 16 - CTK
        k0 = jnp.maximum(_trunc_pos(jnp.sum(jnp.where(tk, bia, 0.0))), 1)
        k1 = jnp.maximum(_trunc_pos(jnp.sum(jnp.where(tk, bib, 0.0))), 1)
        tbuf[...] = jnp.full((16,), 1.0, jnp.float32) * k0.astype(jnp.float32)
        pltpu.sync_copy(tbuf, k_hbm.at[g0])
        tbuf[...] = jnp.full((16,), 1.0, jnp.float32) * k1.astype(jnp.float32)
        pltpu.sync_copy(tbuf, k_hbm.at[g1])

    return sck(iou_t)


def _sc_thresholds(cost_t, kk):
    mesh = plsc.VectorSubcoreMesh(core_axis_name="c", subcore_axis_name="s")

    @functools.partial(
        pl.kernel,
        mesh=mesh,
        compiler_params=pltpu.CompilerParams(needs_layout_passes=False),
        out_type=jax.ShapeDtypeStruct((G, 16), jnp.float32),
        scratch_types=[
            pltpu.VMEM((_NROW, 128), jnp.float32),
            pltpu.VMEM((_NROW, 128), jnp.float32),
            pltpu.VMEM((16,), jnp.float32),
            pltpu.VMEM((16,), jnp.float32),
            pltpu.VMEM((16,), jnp.float32),
            pltpu.SemaphoreType.DMA,
            pltpu.SemaphoreType.DMA,
        ],
    )
    def sck(cost_hbm, k_hbm, t_hbm, bc0, bc1, tbuf, kb0, kb1, s0, s1):
        wid = lax.axis_index("s") * 2 + lax.axis_index("c")
        g0 = wid
        g1 = wid + 32
        d0 = pltpu.async_copy(cost_hbm.at[g0], bc0, s0)
        d1 = pltpu.async_copy(cost_hbm.at[g1], bc1, s1)
        pltpu.sync_copy(k_hbm.at[g0], kb0)
        pltpu.sync_copy(k_hbm.at[g1], kb1)
        k0 = jnp.max(kb0[...]).astype(jnp.int32)
        k1 = jnp.max(kb1[...]).astype(jnp.int32)
        d0.wait()
        d1.wait()
        bca, bcb = _dual_scan(bc0, bc1, CTK, False)
        io16 = jax.lax.iota(jnp.int32, 16)
        t0 = jnp.max(jnp.where(io16 == k0 - 1, bca, NEG_BIG))
        t1 = jnp.max(jnp.where(io16 == k1 - 1, bcb, NEG_BIG))
        tbuf[...] = jnp.full((16,), 1.0, jnp.float32) * t0
        pltpu.sync_copy(tbuf, t_hbm.at[g0])
        tbuf[...] = jnp.full((16,), 1.0, jnp.float32) * t1
        pltpu.sync_copy(tbuf, t_hbm.at[g1])

    return sck(cost_t, kk)


def _assign_kernel(cost_ref, iou_ref, t_ref, gi_ref, lab_ref, giou_ref):
    x = cost_ref[...]
    io = iou_ref[...]
    t = t_ref[...].reshape(G, 1, 1)
    sel = x <= t
    nsel = jnp.sum(sel.astype(jnp.int32), axis=0, keepdims=True)
    gio = jax.lax.broadcasted_iota(jnp.int32, (G, BNR, 128), 0)
    colmin = jnp.min(x, axis=0, keepdims=True)
    amin = jnp.min(jnp.where(x == colmin, gio, 2 ** 30), axis=0,
                   keepdims=True)
    g1 = jnp.min(jnp.where(sel, gio, 2 ** 30), axis=0, keepdims=True)
    assigned = jnp.where(nsel >= 2, amin, g1)
    fg = nsel > 0
    iou_sel = jnp.sum(jnp.where(gio == assigned, io, 0.0), axis=0,
                      keepdims=True)
    gi_ref[...] = jnp.where(fg, assigned, 0)
    lab_ref[...] = jnp.where(fg, 1, -1)
    giou_ref[...] = jnp.where(fg, iou_sel, -INF)


def _pass_b(cost_t, iou_t, t_col):
    return pl.pallas_call(
        _assign_kernel,
        grid=(NP // BN,),
        in_specs=[
            pl.BlockSpec((G, BNR, 128), lambda i: (0, i, 0)),
            pl.BlockSpec((G, BNR, 128), lambda i: (0, i, 0)),
            pl.BlockSpec((G, 1), lambda i: (0, 0)),
        ],
        out_specs=[
            pl.BlockSpec((1, BNR, 128), lambda i: (0, i, 0)),
            pl.BlockSpec((1, BNR, 128), lambda i: (0, i, 0)),
            pl.BlockSpec((1, BNR, 128), lambda i: (0, i, 0)),
        ],
        out_shape=[
            jax.ShapeDtypeStruct((1, NP // 128, 128), jnp.int32),
            jax.ShapeDtypeStruct((1, NP // 128, 128), jnp.int32),
            jax.ShapeDtypeStruct((1, NP // 128, 128), jnp.float32),
        ],
    )(cost_t, iou_t, t_col)


@jax.jit
def kernel(pred_scores, priors, decoded_bboxes, gt_bboxes, gt_labels):
    p_pad, pr_t, db_t = _prep(pred_scores, priors, decoded_bboxes)
    iou_t = _pass_a1(pr_t, db_t, gt_bboxes)
    kk = _sc_dynk(iou_t)
    cost_t = _pass_a2(p_pad, pr_t, db_t, gt_bboxes, gt_labels)
    t16 = _sc_thresholds(cost_t, kk)
    gi, lab, giou = _pass_b(cost_t, iou_t, t16[:, 0:1])
    return (gi.reshape(NP)[:N], lab.reshape(NP)[:N], giou.reshape(NP)[:N])
